# Initial kernel scaffold; baseline (speedup 1.0000x reference)
import jax
import jax.numpy as jnp
from jax.experimental import pallas as pl


def kernel(lgn_spikes, bkg_spikes, lgn_rows, lgn_cols, lgn_weights, bkg_rows, bkg_cols, bkg_weights):
    raise NotImplementedError("write your pallas kernel here")


# speedup vs baseline: 1.7846x; 1.7846x over previous
import functools

import jax
import jax.numpy as jnp
from jax import lax
from jax.experimental import pallas as pl
from jax.experimental.pallas import tpu as pltpu
from jax.experimental.pallas import tpu_sc as plsc

C1 = 17400
C2 = 100
C = C1 + C2
OUT = 40000
OUTP = 40960
T = 100
TPAD = 128
W = 32
NCHUNK = TPAD // W
NT = 16
E = 128
NE_RAW = 500000 + 160000
NB = -(-NE_RAW // (NT * E))
EPT = NB * E
NE = NT * EPT
ROWS_PER_TILE = OUTP // NT
OBUF_ROWS = 320


def _body(tab, rows, cols, ws, zblk, out,
          colv, rowv, wv, gath, contrib, obuf, accum, sem):
    c = lax.axis_index("c")
    s = lax.axis_index("s")
    r0 = s * ROWS_PER_TILE

    for p in range(2):
        chunk = 2 * c + p
        choff = chunk * C
        pltpu.sync_copy(zblk, accum.at[pl.ds(r0, ROWS_PER_TILE)])
        plsc.subcore_barrier()

        def block(b, _):
            base = s * EPT + b * E
            pltpu.sync_copy(cols.at[pl.ds(base, E)], colv)
            pltpu.sync_copy(rows.at[pl.ds(base, E)], rowv)
            pltpu.sync_copy(ws.at[pl.ds(base, E)], wv)

            def addoff(i, _):
                colv[pl.ds(i * 16, 16)] = colv[pl.ds(i * 16, 16)] + choff
                return 0
            lax.fori_loop(0, E // 16, addoff, 0)

            pltpu.async_copy(tab.at[colv], gath, sem).wait()

            def group(g, _):
                w16 = wv[pl.ds(g * 16, 16)]
                for j in range(16):
                    e = g * 16 + j
                    wsc = w16[j]
                    contrib[e, pl.ds(0, 16)] = wsc * gath[e, pl.ds(0, 16)]
                    contrib[e, pl.ds(16, 16)] = wsc * gath[e, pl.ds(16, 16)]
                return 0
            lax.fori_loop(0, E // 16, group, 0)

            pltpu.sync_copy(contrib, accum.at[rowv], add=True)
            return 0

        lax.fori_loop(0, NB, block, 0)
        plsc.subcore_barrier()

        def wout(k, _):
            rb = r0 + k * OBUF_ROWS
            pltpu.sync_copy(accum.at[pl.ds(rb, OBUF_ROWS)], obuf)
            pltpu.sync_copy(obuf, out.at[pl.ds(chunk * OUTP + rb, OBUF_ROWS)])
            return 0
        lax.fori_loop(0, ROWS_PER_TILE // OBUF_ROWS, wout, 0)
        plsc.subcore_barrier()


@functools.partial(
    pl.kernel,
    out_type=jax.ShapeDtypeStruct((NCHUNK * OUTP, W), jnp.float32),
    mesh=plsc.VectorSubcoreMesh(core_axis_name="c", subcore_axis_name="s"),
    scratch_types=[
        pltpu.VMEM((E,), jnp.int32),
        pltpu.VMEM((E,), jnp.int32),
        pltpu.VMEM((E,), jnp.float32),
        pltpu.VMEM((E, W), jnp.float32),
        pltpu.VMEM((E, W), jnp.float32),
        pltpu.VMEM((OBUF_ROWS, W), jnp.float32),
        pltpu.VMEM_SHARED((OUTP, W), jnp.float32),
        pltpu.SemaphoreType.DMA,
    ],
    compiler_params=pltpu.CompilerParams(use_tc_tiling_on_sc=False),
)
def _sc_kernel(tab, rows, cols, ws, zblk, out, *scratch):
    _body(tab, rows, cols, ws, zblk, out, *scratch)


def kernel(lgn_spikes, bkg_spikes, lgn_rows, lgn_cols, lgn_weights,
           bkg_rows, bkg_cols, bkg_weights):
    spikes = jnp.concatenate(
        [lgn_spikes.reshape(T, C1), bkg_spikes.reshape(T, C2)], axis=1)
    spikes = jnp.pad(spikes, ((0, TPAD - T), (0, 0)))
    tab = spikes.T.reshape(C, NCHUNK, W)
    tab = tab.transpose(1, 0, 2).reshape(NCHUNK * C, W)

    rows = jnp.concatenate([lgn_rows, bkg_rows])
    cols = jnp.concatenate([lgn_cols, bkg_cols + C1])
    ws = jnp.concatenate([lgn_weights, bkg_weights])
    pad = NE - NE_RAW
    rows = jnp.pad(rows, (0, pad))
    cols = jnp.pad(cols, (0, pad))
    ws = jnp.pad(ws, (0, pad))

    zblk = jnp.zeros((ROWS_PER_TILE, W), jnp.float32)

    out = _sc_kernel(tab, rows, cols, ws, zblk)

    cur = out.reshape(NCHUNK, OUTP, W)[:, :OUT].transpose(0, 2, 1).reshape(TPAD, OUT)
    return cur[:T][None]


# speedup vs baseline: 3.3038x; 1.8513x over previous
import functools

import jax
import jax.numpy as jnp
from jax import lax
from jax.experimental import pallas as pl
from jax.experimental.pallas import tpu as pltpu
from jax.experimental.pallas import tpu_sc as plsc

C1 = 17400
C2 = 100
C = C1 + C2
OUT = 40000
OUTP = 40960
T = 100
TPAD = 128
W = 64
NCHUNK = TPAD // W
NT = 16
E = 128
NE_RAW = 500000 + 160000
NB = -(-NE_RAW // (NT * E))
EPT = NB * E
NE = NT * EPT
ROWS_PER_TILE = OUTP // NT
OBUF_ROWS = 320


def _body(tab, rows, cols, ws, zblk, out,
          colv, rowv, wv, gath, contrib, obuf, accum, sem):
    c = lax.axis_index("c")
    s = lax.axis_index("s")
    r0 = s * ROWS_PER_TILE
    choff = c * C

    pltpu.sync_copy(zblk, accum.at[pl.ds(r0, ROWS_PER_TILE)])
    plsc.subcore_barrier()

    def block(b, _):
        base = s * EPT + b * E
        pltpu.sync_copy(cols.at[pl.ds(base, E)], colv)
        pltpu.sync_copy(rows.at[pl.ds(base, E)], rowv)
        pltpu.sync_copy(ws.at[pl.ds(base, E)], wv)

        def addoff(i, _):
            colv[pl.ds(i * 16, 16)] = colv[pl.ds(i * 16, 16)] + choff
            return 0
        lax.fori_loop(0, E // 16, addoff, 0)

        pltpu.async_copy(tab.at[colv], gath, sem).wait()

        def group(g, _):
            w16 = wv[pl.ds(g * 16, 16)]
            for j in range(16):
                e = g * 16 + j
                wbc = jnp.full((16,), w16[j], jnp.float32)
                wbf = plsc.pack(wbc, wbc, format=plsc.PackFormat.INTERLEAVED)
                contrib[e, pl.ds(0, 32)] = wbf * gath[e, pl.ds(0, 32)]
                contrib[e, pl.ds(32, 32)] = wbf * gath[e, pl.ds(32, 32)]
            return 0
        lax.fori_loop(0, E // 16, group, 0)

        pltpu.sync_copy(contrib, accum.at[rowv], add=True)
        return 0

    lax.fori_loop(0, NB, block, 0)
    plsc.subcore_barrier()

    def wout(k, _):
        rb = r0 + k * OBUF_ROWS
        pltpu.sync_copy(accum.at[pl.ds(rb, OBUF_ROWS)], obuf)
        pltpu.sync_copy(obuf, out.at[pl.ds(c * OUTP + rb, OBUF_ROWS)])
        return 0
    lax.fori_loop(0, ROWS_PER_TILE // OBUF_ROWS, wout, 0)


@functools.partial(
    pl.kernel,
    out_type=jax.ShapeDtypeStruct((NCHUNK * OUTP, W), jnp.bfloat16),
    mesh=plsc.VectorSubcoreMesh(core_axis_name="c", subcore_axis_name="s"),
    scratch_types=[
        pltpu.VMEM((E,), jnp.int32),
        pltpu.VMEM((E,), jnp.int32),
        pltpu.VMEM((E,), jnp.float32),
        pltpu.VMEM((E, W), jnp.bfloat16),
        pltpu.VMEM((E, W), jnp.bfloat16),
        pltpu.VMEM((OBUF_ROWS, W), jnp.bfloat16),
        pltpu.VMEM_SHARED((OUTP, W), jnp.bfloat16),
        pltpu.SemaphoreType.DMA,
    ],
    compiler_params=pltpu.CompilerParams(
        use_tc_tiling_on_sc=False, needs_layout_passes=False),
)
def _sc_kernel(tab, rows, cols, ws, zblk, out, *scratch):
    _body(tab, rows, cols, ws, zblk, out, *scratch)


def kernel(lgn_spikes, bkg_spikes, lgn_rows, lgn_cols, lgn_weights,
           bkg_rows, bkg_cols, bkg_weights):
    spikes = jnp.concatenate(
        [lgn_spikes.reshape(T, C1), bkg_spikes.reshape(T, C2)], axis=1)
    spikes = jnp.pad(spikes, ((0, TPAD - T), (0, 0)))
    tab = spikes.T.reshape(C, NCHUNK, W)
    tab = tab.transpose(1, 0, 2).reshape(NCHUNK * C, W)
    tab = tab.astype(jnp.bfloat16)

    rows = jnp.concatenate([lgn_rows, bkg_rows])
    cols = jnp.concatenate([lgn_cols, bkg_cols + C1])
    ws = jnp.concatenate([lgn_weights, bkg_weights])
    pad = NE - NE_RAW
    rows = jnp.pad(rows, (0, pad))
    cols = jnp.pad(cols, (0, pad))
    ws = jnp.pad(ws, (0, pad))

    zblk = jnp.zeros((ROWS_PER_TILE, W), jnp.bfloat16)

    out = _sc_kernel(tab, rows, cols, ws, zblk)

    cur = out.reshape(NCHUNK, OUTP, W)[:, :OUT].transpose(0, 2, 1)
    cur = cur.reshape(TPAD, OUT)[:T].astype(jnp.float32)
    return cur[None]


# speedup vs baseline: 6.6604x; 2.0160x over previous
import functools

import jax
import jax.numpy as jnp
from jax import lax
from jax.experimental import pallas as pl
from jax.experimental.pallas import tpu as pltpu
from jax.experimental.pallas import tpu_sc as plsc

C1 = 17400
C2 = 100
C = C1 + C2
OUT = 40000
OUTP = 40960
T = 100
TPAD = 128
W = 64
NCHUNK = TPAD // W
NT = 16
E = 128
BPC = 4
IDXC = BPC * E
NE_RAW = 500000 + 160000
NC3 = 27
NCHK = NC3 * 3
EPT = NCHK * IDXC
NE = NT * EPT
ROWS_PER_TILE = OUTP // NT
OBUF_ROWS = 320


def _compute(wv, gath, contrib, woff):
    def group(g, _):
        w16 = wv[woff, pl.ds(g * 16, 16)]
        for j in range(16):
            e = g * 16 + j
            wbc = jnp.full((16,), w16[j], jnp.float32)
            wbf = plsc.pack(wbc, wbc, format=plsc.PackFormat.INTERLEAVED)
            contrib[e, pl.ds(0, 32)] = wbf * gath[e, pl.ds(0, 32)]
            contrib[e, pl.ds(32, 32)] = wbf * gath[e, pl.ds(32, 32)]
        return 0
    lax.fori_loop(0, E // 16, group, 0)


def _body(tab, rows2, cols2, ws2, zblk, out, *sc):
    (colv0, colv1, colv2, rowv0, rowv1, rowv2, wv0, wv1, wv2,
     gathA, gathB, contribA, contribB, obuf, accum,
     isem0, isem1, isem2, gsemA, gsemB, ssemA, ssemB) = sc
    colv = (colv0, colv1, colv2)
    rowv = (rowv0, rowv1, rowv2)
    wv = (wv0, wv1, wv2)
    isem = (isem0, isem1, isem2)
    gath = (gathA, gathB)
    contrib = (contribA, contribB)
    gsem = (gsemA, gsemB)
    ssem = (ssemA, ssemB)

    c = lax.axis_index("c")
    s = lax.axis_index("s")
    r0 = s * ROWS_PER_TILE
    choff = c * C
    crow0 = s * (EPT // E)

    pltpu.sync_copy(zblk, accum.at[pl.ds(r0, ROWS_PER_TILE)])
    plsc.subcore_barrier()

    def load_idx(cidx, buf, sync):
        rb = crow0 + cidx * BPC
        if sync:
            pltpu.sync_copy(cols2.at[pl.ds(rb, BPC)], colv[buf])
            pltpu.sync_copy(rows2.at[pl.ds(rb, BPC)], rowv[buf])
            pltpu.sync_copy(ws2.at[pl.ds(rb, BPC)], wv[buf])
        else:
            pltpu.async_copy(cols2.at[pl.ds(rb, BPC)], colv[buf], isem[buf])
            pltpu.async_copy(rows2.at[pl.ds(rb, BPC)], rowv[buf], isem[buf])
            pltpu.async_copy(ws2.at[pl.ds(rb, BPC)], wv[buf], isem[buf])

    def wait_idx(buf):
        pltpu.make_async_copy(cols2.at[pl.ds(0, BPC)], colv[buf], isem[buf]).wait()
        pltpu.make_async_copy(rows2.at[pl.ds(0, BPC)], rowv[buf], isem[buf]).wait()
        pltpu.make_async_copy(ws2.at[pl.ds(0, BPC)], wv[buf], isem[buf]).wait()

    def adjust_cols(buf):
        for r in range(BPC):
            def addoff_r(i, _, r=r):
                colv[buf][r, pl.ds(i * 16, 16)] = \
                    colv[buf][r, pl.ds(i * 16, 16)] + choff
                return 0
            lax.fori_loop(0, E // 16, addoff_r, 0)

    def issue_gather(buf, b, gpar):
        pltpu.async_copy(tab.at[colv[buf].at[b]], gath[gpar], gsem[gpar])

    def wait_gather(gpar):
        pltpu.make_async_copy(tab.at[colv[0].at[0]], gath[gpar],
                              gsem[gpar]).wait()

    def issue_scatter(buf, b, spar):
        pltpu.async_copy(contrib[spar], accum.at[rowv[buf].at[b]],
                         ssem[spar], add=True)

    def wait_scatter(spar):
        pltpu.make_async_copy(contrib[spar], accum.at[rowv[0].at[0]],
                              ssem[spar]).wait()

    load_idx(0, 0, sync=True)
    adjust_cols(0)
    load_idx(1, 1, sync=False)
    issue_gather(0, 0, 0)

    def triple(k3, _):
        for q in range(3):
            cidx = 3 * k3 + q
            for b in range(BPC):
                gpar = b % 2
                if b == 2:
                    if q == 0:
                        load_idx(cidx + 2, (q + 2) % 3, sync=False)
                    else:
                        @pl.when(k3 < NC3 - 1)
                        def _():
                            load_idx(cidx + 2, (q + 2) % 3, sync=False)
                wait_gather(gpar)
                if b < BPC - 1:
                    issue_gather(q, b + 1, 1 - gpar)
                else:
                    nq = (q + 1) % 3
                    if q == 2:
                        @pl.when(k3 < NC3 - 1)
                        def _():
                            wait_idx(nq)
                            adjust_cols(nq)
                            issue_gather(nq, 0, 1 - gpar)
                    else:
                        wait_idx(nq)
                        adjust_cols(nq)
                        issue_gather(nq, 0, 1 - gpar)
                if q == 0 and b < 2:
                    @pl.when(k3 > 0)
                    def _():
                        wait_scatter(gpar)
                else:
                    wait_scatter(gpar)
                _compute(wv[q], gath[gpar], contrib[gpar], b)
                issue_scatter(q, b, gpar)
        return 0

    lax.fori_loop(0, NC3, triple, 0)

    wait_scatter(0)
    wait_scatter(1)
    plsc.subcore_barrier()

    def wout(k, _):
        rb = r0 + k * OBUF_ROWS
        pltpu.sync_copy(accum.at[pl.ds(rb, OBUF_ROWS)], obuf)
        pltpu.sync_copy(obuf, out.at[pl.ds(c * OUTP + rb, OBUF_ROWS)])
        return 0
    lax.fori_loop(0, ROWS_PER_TILE // OBUF_ROWS, wout, 0)


@functools.partial(
    pl.kernel,
    out_type=jax.ShapeDtypeStruct((NCHUNK * OUTP, W), jnp.bfloat16),
    mesh=plsc.VectorSubcoreMesh(core_axis_name="c", subcore_axis_name="s"),
    scratch_types=(
        [pltpu.VMEM((BPC, E), jnp.int32) for _ in range(3)]
        + [pltpu.VMEM((BPC, E), jnp.int32) for _ in range(3)]
        + [pltpu.VMEM((BPC, E), jnp.float32) for _ in range(3)]
        + [pltpu.VMEM((E, W), jnp.bfloat16) for _ in range(2)]
        + [pltpu.VMEM((E, W), jnp.bfloat16) for _ in range(2)]
        + [
            pltpu.VMEM((OBUF_ROWS, W), jnp.bfloat16),
            pltpu.VMEM_SHARED((OUTP, W), jnp.bfloat16),
        ]
        + [pltpu.SemaphoreType.DMA for _ in range(7)]
    ),
    compiler_params=pltpu.CompilerParams(
        use_tc_tiling_on_sc=False, needs_layout_passes=False),
)
def _sc_kernel(tab, rows2, cols2, ws2, zblk, out, *scratch):
    _body(tab, rows2, cols2, ws2, zblk, out, *scratch)


def kernel(lgn_spikes, bkg_spikes, lgn_rows, lgn_cols, lgn_weights,
           bkg_rows, bkg_cols, bkg_weights):
    spikes = jnp.concatenate(
        [lgn_spikes.reshape(T, C1), bkg_spikes.reshape(T, C2)], axis=1)
    spikes = jnp.pad(spikes, ((0, TPAD - T), (0, 0)))
    tab = spikes.T.reshape(C, NCHUNK, W)
    tab = tab.transpose(1, 0, 2).reshape(NCHUNK * C, W)
    tab = tab.astype(jnp.bfloat16)

    rows = jnp.concatenate([lgn_rows, bkg_rows])
    cols = jnp.concatenate([lgn_cols, bkg_cols + C1])
    ws = jnp.concatenate([lgn_weights, bkg_weights])
    pad = NE - NE_RAW
    rows2 = jnp.pad(rows, (0, pad)).reshape(NE // E, E)
    cols2 = jnp.pad(cols, (0, pad)).reshape(NE // E, E)
    ws2 = jnp.pad(ws, (0, pad)).reshape(NE // E, E)

    zblk = jnp.zeros((ROWS_PER_TILE, W), jnp.bfloat16)

    out = _sc_kernel(tab, rows2, cols2, ws2, zblk)

    cur = out.reshape(NCHUNK, OUTP, W)[:, :OUT].transpose(0, 2, 1)
    cur = cur.reshape(TPAD, OUT)[:T].astype(jnp.float32)
    return cur[None]


# speedup vs baseline: 6.8908x; 1.0346x over previous
import functools

import jax
import jax.numpy as jnp
from jax import lax
from jax.experimental import pallas as pl
from jax.experimental.pallas import tpu as pltpu
from jax.experimental.pallas import tpu_sc as plsc

C1 = 17400
C2 = 100
C = C1 + C2
OUT = 40000
OUTP = 40960
T = 100
TPAD = 128
W = 64
NCHUNK = TPAD // W
NT = 16
E = 128
BPC = 4
IDXC = BPC * E
NE_RAW = 500000 + 160000
NC3 = 27
NCHK = NC3 * 3
EPT = NCHK * IDXC
NE = NT * EPT
ROWS_PER_TILE = OUTP // NT
OBUF_ROWS = 320


def _compute(wv, gath, contrib, woff):
    def group(g, _):
        w16 = wv[woff, pl.ds(g * 16, 16)]
        for j in range(16):
            e = g * 16 + j
            wbc = jnp.full((16,), w16[j], jnp.float32)
            wbf = plsc.pack(wbc, wbc, format=plsc.PackFormat.INTERLEAVED)
            contrib[e, pl.ds(0, 32)] = wbf * gath[e, pl.ds(0, 32)]
            contrib[e, pl.ds(32, 32)] = wbf * gath[e, pl.ds(32, 32)]
        return 0
    lax.fori_loop(0, E // 16, group, 0)


def _body(tab, rows2, cols2, ws2, zblk, out, *sc):
    (colv0, colv1, colv2, rowv0, rowv1, rowv2, wv0, wv1, wv2,
     gathA, gathB, contribA, contribB, obuf, accum,
     isem0, isem1, isem2, gsemA, gsemB, ssemA, ssemB) = sc
    colv = (colv0, colv1, colv2)
    rowv = (rowv0, rowv1, rowv2)
    wv = (wv0, wv1, wv2)
    isem = (isem0, isem1, isem2)
    gath = (gathA, gathB)
    contrib = (contribA, contribB)
    gsem = (gsemA, gsemB)
    ssem = (ssemA, ssemB)

    c = lax.axis_index("c")
    s = lax.axis_index("s")
    r0 = s * ROWS_PER_TILE
    choff = c * C
    crow0 = s * (EPT // E)

    pltpu.sync_copy(zblk, accum.at[pl.ds(r0, ROWS_PER_TILE)])
    plsc.subcore_barrier()

    def load_idx(cidx, buf, sync):
        rb = crow0 + cidx * BPC
        if sync:
            pltpu.sync_copy(cols2.at[pl.ds(rb, BPC)], colv[buf])
            pltpu.sync_copy(rows2.at[pl.ds(rb, BPC)], rowv[buf])
            pltpu.sync_copy(ws2.at[pl.ds(rb, BPC)], wv[buf])
        else:
            pltpu.async_copy(cols2.at[pl.ds(rb, BPC)], colv[buf], isem[buf])
            pltpu.async_copy(rows2.at[pl.ds(rb, BPC)], rowv[buf], isem[buf])
            pltpu.async_copy(ws2.at[pl.ds(rb, BPC)], wv[buf], isem[buf])

    def wait_idx(buf):
        pltpu.make_async_copy(cols2.at[pl.ds(0, BPC)], colv[buf], isem[buf]).wait()
        pltpu.make_async_copy(rows2.at[pl.ds(0, BPC)], rowv[buf], isem[buf]).wait()
        pltpu.make_async_copy(ws2.at[pl.ds(0, BPC)], wv[buf], isem[buf]).wait()

    def adjust_cols(buf):
        for r in range(BPC):
            def addoff_r(i, _, r=r):
                colv[buf][r, pl.ds(i * 16, 16)] = \
                    colv[buf][r, pl.ds(i * 16, 16)] + choff
                return 0
            lax.fori_loop(0, E // 16, addoff_r, 0)

    def issue_gather(buf, b, gpar):
        pltpu.async_copy(tab.at[colv[buf].at[b]], gath[gpar], gsem[gpar])

    def wait_gather(gpar):
        pltpu.make_async_copy(tab.at[colv[0].at[0]], gath[gpar],
                              gsem[gpar]).wait()

    def issue_scatter(buf, b, spar):
        pltpu.async_copy(contrib[spar], accum.at[rowv[buf].at[b]],
                         ssem[spar], add=True)

    def wait_scatter(spar):
        pltpu.make_async_copy(contrib[spar], accum.at[rowv[0].at[0]],
                              ssem[spar]).wait()

    load_idx(0, 0, sync=True)
    adjust_cols(0)
    load_idx(1, 1, sync=False)
    issue_gather(0, 0, 0)

    def triple(k3, _):
        for q in range(3):
            cidx = 3 * k3 + q
            for b in range(BPC):
                gpar = b % 2
                if b == 2:
                    if q == 0:
                        load_idx(cidx + 2, (q + 2) % 3, sync=False)
                    else:
                        @pl.when(k3 < NC3 - 1)
                        def _():
                            load_idx(cidx + 2, (q + 2) % 3, sync=False)
                wait_gather(gpar)
                if b < BPC - 1:
                    issue_gather(q, b + 1, 1 - gpar)
                else:
                    nq = (q + 1) % 3
                    if q == 2:
                        @pl.when(k3 < NC3 - 1)
                        def _():
                            wait_idx(nq)
                            adjust_cols(nq)
                            issue_gather(nq, 0, 1 - gpar)
                    else:
                        wait_idx(nq)
                        adjust_cols(nq)
                        issue_gather(nq, 0, 1 - gpar)
                if q == 0 and b < 2:
                    @pl.when(k3 > 0)
                    def _():
                        wait_scatter(gpar)
                else:
                    wait_scatter(gpar)
                _compute(wv[q], gath[gpar], contrib[gpar], b)
                issue_scatter(q, b, gpar)
        return 0

    lax.fori_loop(0, NC3, triple, 0)

    wait_scatter(0)
    wait_scatter(1)
    plsc.subcore_barrier()

    def wout(k, _):
        rb = r0 + k * OBUF_ROWS
        pltpu.sync_copy(accum.at[pl.ds(rb, OBUF_ROWS)], obuf)
        pltpu.sync_copy(obuf, out.at[pl.ds(c * OUTP + rb, OBUF_ROWS)])
        return 0
    lax.fori_loop(0, ROWS_PER_TILE // OBUF_ROWS, wout, 0)


@functools.partial(
    pl.kernel,
    out_type=jax.ShapeDtypeStruct((NCHUNK * OUTP, W), jnp.bfloat16),
    mesh=plsc.VectorSubcoreMesh(core_axis_name="c", subcore_axis_name="s"),
    scratch_types=(
        [pltpu.VMEM((BPC, E), jnp.int32) for _ in range(3)]
        + [pltpu.VMEM((BPC, E), jnp.int32) for _ in range(3)]
        + [pltpu.VMEM((BPC, E), jnp.float32) for _ in range(3)]
        + [pltpu.VMEM((E, W), jnp.bfloat16) for _ in range(2)]
        + [pltpu.VMEM((E, W), jnp.bfloat16) for _ in range(2)]
        + [
            pltpu.VMEM((OBUF_ROWS, W), jnp.bfloat16),
            pltpu.VMEM_SHARED((OUTP, W), jnp.bfloat16),
        ]
        + [pltpu.SemaphoreType.DMA for _ in range(7)]
    ),
    compiler_params=pltpu.CompilerParams(
        use_tc_tiling_on_sc=False, needs_layout_passes=False),
)
def _sc_kernel(tab, rows2, cols2, ws2, zblk, out, *scratch):
    _body(tab, rows2, cols2, ws2, zblk, out, *scratch)


def kernel(lgn_spikes, bkg_spikes, lgn_rows, lgn_cols, lgn_weights,
           bkg_rows, bkg_cols, bkg_weights):
    spikes = jnp.concatenate(
        [lgn_spikes.reshape(T, C1), bkg_spikes.reshape(T, C2)], axis=1)
    spikes = jnp.pad(spikes.astype(jnp.bfloat16), ((0, TPAD - T), (0, 0)))
    tab = spikes.reshape(NCHUNK, W, C).transpose(0, 2, 1)
    tab = tab.reshape(NCHUNK * C, W)

    rows = jnp.concatenate([lgn_rows, bkg_rows])
    cols = jnp.concatenate([lgn_cols, bkg_cols + C1])
    ws = jnp.concatenate([lgn_weights, bkg_weights])
    pad = NE - NE_RAW
    rows2 = jnp.pad(rows, (0, pad)).reshape(NE // E, E)
    cols2 = jnp.pad(cols, (0, pad)).reshape(NE // E, E)
    ws2 = jnp.pad(ws, (0, pad)).reshape(NE // E, E)

    zblk = jnp.zeros((ROWS_PER_TILE, W), jnp.bfloat16)

    out = _sc_kernel(tab, rows2, cols2, ws2, zblk)

    cur = out.reshape(NCHUNK, OUTP, W).transpose(0, 2, 1).reshape(TPAD, OUTP)
    return cur[:T, :OUT].astype(jnp.float32)[None]
